# Initial kernel scaffold; baseline (speedup 1.0000x reference)
import jax
import jax.numpy as jnp
from jax.experimental import pallas as pl


def kernel(x, edge_index, W):
    raise NotImplementedError("write your pallas kernel here")


# speedup vs baseline: 2.8167x; 2.8167x over previous
import functools

import jax
import jax.numpy as jnp
from jax import lax
from jax.experimental import pallas as pl
from jax.experimental.pallas import tpu as pltpu
from jax.experimental.pallas import tpu_sc as plsc

K_STEPS = 3
ALPHA = 0.5
NS = 16
NC = 2
L = 16
EB = 80
DQ = 64


def _rsqrt16(d):
    half = d * 0.5
    i = plsc.bitcast(d, jnp.int32)
    i = jnp.full((L,), 0x5F3759DF, jnp.int32) - lax.shift_right_arithmetic(
        i, jnp.full((L,), 1, jnp.int32))
    y = plsc.bitcast(i, jnp.float32)
    for _ in range(3):
        y = y * (1.5 - half * y * y)
    return y


def _fill(ref, rows, vec16s, value):
    v = jnp.full((L,), value, jnp.float32)

    def body(r, _):
        for j in range(vec16s):
            ref[r, pl.ds(j * L, L)] = v
        return _

    lax.fori_loop(0, rows, body, None)


def _sc_propagate(n, e, nb, nt, nch, ch):
    vq = DQ // L

    mesh = plsc.VectorSubcoreMesh(core_axis_name="c", subcore_axis_name="s")

    @functools.partial(
        pl.kernel,
        mesh=mesh,
        compiler_params=pltpu.CompilerParams(
            use_tc_tiling_on_sc=False, needs_layout_passes=False),
        out_type=(
            jax.ShapeDtypeStruct((n, 4 * DQ), jnp.float32),
            jax.ShapeDtypeStruct((4 * n, DQ), jnp.float32),
        ),
        scratch_types=dict(
            uacc=pltpu.VMEM_SHARED((n, DQ), jnp.float32),
            dga=pltpu.VMEM_SHARED((n, L), jnp.float32),
            dgb=pltpu.VMEM_SHARED((n, L), jnp.float32),
            srcg0=pltpu.VMEM((nb, EB), jnp.int32),
            dstv=pltpu.VMEM((nb, EB), jnp.int32),
            gbuf=pltpu.VMEM((EB, DQ), jnp.float32),
            ones=pltpu.VMEM((EB, L), jnp.float32),
            snorm=pltpu.VMEM((nt, L), jnp.float32),
            dnorm=pltpu.VMEM((nt, L), jnp.float32),
            uch=pltpu.VMEM((ch, DQ), jnp.float32),
            ych=pltpu.VMEM((ch, DQ), jnp.float32),
        ),
    )
    def prop(y0_hbm, src_hbm, dst_hbm, out_hbm, h_hbm,
             uacc, dga, dgb, srcg0, dstv, gbuf, ones, snorm, dnorm,
             uch, ych):
        c = lax.axis_index("c")
        s = lax.axis_index("s")
        n0 = s * nt

        _fill(uch, ch, vq, 0.0)
        for t in range(nch):
            pltpu.sync_copy(uch, uacc.at[pl.ds(n0 + t * ch, ch)])
        _fill(snorm, nt, 1, 0.0)
        pltpu.sync_copy(snorm, dga.at[pl.ds(n0, nt)])
        pltpu.sync_copy(snorm, dgb.at[pl.ds(n0, nt)])
        _fill(ones, EB, 1, 1.0)
        plsc.subcore_barrier()

        pltpu.sync_copy(src_hbm.at[pl.ds(s * nb, nb)], srcg0)
        pltpu.sync_copy(dst_hbm.at[pl.ds(s * nb, nb)], dstv)

        def deg_body(j, _):
            pltpu.sync_copy(ones, dga.at[srcg0.at[j]], add=True)
            pltpu.sync_copy(ones, dgb.at[dstv.at[j]], add=True)
            return _

        lax.fori_loop(0, nb, deg_body, None)

        off0 = jnp.full((L,), 2 * c * n, jnp.int32)

        def _shift(delta):
            dv = jnp.full((L,), delta, jnp.int32)

            def body(j, _):
                for v in range(EB // L):
                    sl = pl.ds(v * L, L)
                    srcg0[j, sl] = srcg0[j, sl] + dv
                return _

            lax.fori_loop(0, nb, body, None)

        def off_body(j, _):
            for v in range(EB // L):
                sl = pl.ds(v * L, L)
                srcg0[j, sl] = srcg0[j, sl] + off0
            return _

        lax.fori_loop(0, nb, off_body, None)
        plsc.subcore_barrier()

        pltpu.sync_copy(dga.at[pl.ds(n0, nt)], snorm)
        pltpu.sync_copy(dgb.at[pl.ds(n0, nt)], dnorm)

        def norm_body(r, _):
            sl = pl.ds(0, L)
            snorm[r, sl] = _rsqrt16(jnp.maximum(snorm[r, sl], 1.0))
            dnorm[r, sl] = _rsqrt16(jnp.maximum(dnorm[r, sl], 1.0))
            return _

        lax.fori_loop(0, nt, norm_body, None)

        for p in range(2):
            yq = (2 * c + p) * n + n0
            for t in range(nch):
                pltpu.sync_copy(y0_hbm.at[pl.ds(yq + t * ch, ch)], ych)

                def h0_body(r, _):
                    sn = snorm[t * ch + r, pl.ds(0, L)]
                    for v in range(vq):
                        sl = pl.ds(v * L, L)
                        ych[r, sl] = ych[r, sl] * sn
                    return _

                lax.fori_loop(0, ch, h0_body, None)
                pltpu.sync_copy(ych, h_hbm.at[pl.ds(yq + t * ch, ch)])
        plsc.subcore_barrier()

        for k in range(K_STEPS):
            last = k == K_STEPS - 1
            for p in range(2):
                if p == 1:
                    _shift(n)

                def edge_body(j, _):
                    pltpu.sync_copy(h_hbm.at[srcg0.at[j]], gbuf)
                    pltpu.sync_copy(gbuf, uacc.at[dstv.at[j]], add=True)
                    return _

                lax.fori_loop(0, nb, edge_body, None)
                if p == 1:
                    _shift(-n)
                plsc.subcore_barrier()

                for t in range(nch):
                    g0 = n0 + t * ch
                    yq = (2 * c + p) * n + g0
                    pltpu.sync_copy(uacc.at[pl.ds(g0, ch)], uch)
                    pltpu.sync_copy(y0_hbm.at[pl.ds(yq, ch)], ych)

                    def dense_body(r, _):
                        sn = snorm[t * ch + r, pl.ds(0, L)]
                        dn = dnorm[t * ch + r, pl.ds(0, L)]
                        if last:
                            a = (1.0 - ALPHA) * dn
                            b = jnp.full((L,), ALPHA, jnp.float32)
                        else:
                            a = (1.0 - ALPHA) * sn * dn
                            b = ALPHA * sn
                        for v in range(vq):
                            sl = pl.ds(v * L, L)
                            uch[r, sl] = a * uch[r, sl] + b * ych[r, sl]
                        return _

                    lax.fori_loop(0, ch, dense_body, None)
                    if last:
                        pltpu.sync_copy(
                            uch,
                            out_hbm.at[pl.ds(g0, ch),
                                       pl.ds((2 * c + p) * DQ, DQ)])
                    else:
                        pltpu.sync_copy(uch, h_hbm.at[pl.ds(yq, ch)])
                    if not (last and p == 1):
                        _fill(ych, ch, vq, 0.0)
                        pltpu.sync_copy(ych, uacc.at[pl.ds(g0, ch)])
                if not (last and p == 1):
                    plsc.subcore_barrier()

    return prop


def _tc_matmul(n, d, bn):
    def body(x_ref, w_ref, o_ref):
        o_ref[...] = lax.dot_general(
            x_ref[...], w_ref[...], (((1,), (1,)), ((), ())),
            preferred_element_type=jnp.float32)

    nblk = n // bn
    return pl.pallas_call(
        body,
        grid=(4, nblk),
        in_specs=[
            pl.BlockSpec((bn, d), lambda q, i: (i, 0)),
            pl.BlockSpec((DQ, d), lambda q, i: (q, 0)),
        ],
        out_specs=pl.BlockSpec((bn, DQ), lambda q, i: (q * nblk + i, 0)),
        out_shape=jax.ShapeDtypeStruct((4 * n, DQ), jnp.float32),
    )


def kernel(x, edge_index, W):
    n, d = x.shape
    e = edge_index.shape[1]

    src = edge_index[0].astype(jnp.int32)
    dst = edge_index[1].astype(jnp.int32)
    nb = e // (NS * EB)
    src2 = src.reshape(NS * nb, EB)
    dst2 = dst.reshape(NS * nb, EB)

    nt = n // NS
    ch = 125
    nch = nt // ch

    y0 = _tc_matmul(n, d, bn=400)(x, W)
    out, _ = _sc_propagate(n, e, nb, nt, nch, ch)(y0, src2, dst2)
    return out


# speedup vs baseline: 4.1879x; 1.4868x over previous
import functools

import jax
import jax.numpy as jnp
from jax import lax
from jax.experimental import pallas as pl
from jax.experimental.pallas import tpu as pltpu
from jax.experimental.pallas import tpu_sc as plsc

K_STEPS = 3
ALPHA = 0.5
NS = 16
NC = 2
L = 16
EB = 80
DQ = 64


def _rsqrt16(d):
    half = d * 0.5
    i = plsc.bitcast(d, jnp.int32)
    i = jnp.full((L,), 0x5F3759DF, jnp.int32) - lax.shift_right_arithmetic(
        i, jnp.full((L,), 1, jnp.int32))
    y = plsc.bitcast(i, jnp.float32)
    for _ in range(3):
        y = y * (1.5 - half * y * y)
    return y


def _fill(ref, rows, vec16s, value):
    v = jnp.full((L,), value, jnp.float32)

    def body(r, _):
        for j in range(vec16s):
            ref[r, pl.ds(j * L, L)] = v
        return _

    lax.fori_loop(0, rows, body, None)


def _sc_propagate(n, e, nb, nt, nch, ch):
    vq = DQ // L

    mesh = plsc.VectorSubcoreMesh(core_axis_name="c", subcore_axis_name="s")

    @functools.partial(
        pl.kernel,
        mesh=mesh,
        compiler_params=pltpu.CompilerParams(
            use_tc_tiling_on_sc=False, needs_layout_passes=False),
        out_type=(
            jax.ShapeDtypeStruct((n, 4 * DQ), jnp.float32),
            jax.ShapeDtypeStruct((4 * n, DQ), jnp.float32),
        ),
        scratch_types=dict(
            uacc=pltpu.VMEM_SHARED((n, DQ), jnp.float32),
            dga=pltpu.VMEM_SHARED((n, L), jnp.float32),
            dgb=pltpu.VMEM_SHARED((n, L), jnp.float32),
            srcg0=pltpu.VMEM((nb, EB), jnp.int32),
            dstv=pltpu.VMEM((nb, EB), jnp.int32),
            gbuf0=pltpu.VMEM((EB, DQ), jnp.float32),
            gbuf1=pltpu.VMEM((EB, DQ), jnp.float32),
            ones=pltpu.VMEM((EB, L), jnp.float32),
            snorm=pltpu.VMEM((nt, L), jnp.float32),
            dnorm=pltpu.VMEM((nt, L), jnp.float32),
            uch=pltpu.VMEM((ch, DQ), jnp.float32),
            ych=pltpu.VMEM((ch, DQ), jnp.float32),
            gs0=pltpu.SemaphoreType.DMA,
            gs1=pltpu.SemaphoreType.DMA,
            ss0=pltpu.SemaphoreType.DMA,
            ss1=pltpu.SemaphoreType.DMA,
        ),
    )
    def prop(y0_hbm, src_hbm, dst_hbm, out_hbm, h_hbm,
             uacc, dga, dgb, srcg0, dstv, gbuf0, gbuf1, ones, snorm, dnorm,
             uch, ych, gs0, gs1, ss0, ss1):
        c = lax.axis_index("c")
        s = lax.axis_index("s")
        n0 = s * nt

        _fill(ych, ch, vq, 0.0)
        for t in range(nch):
            pltpu.sync_copy(ych, uacc.at[pl.ds(n0 + t * ch, ch)])
        _fill(snorm, nt, 1, 0.0)
        pltpu.sync_copy(snorm, dga.at[pl.ds(n0, nt)])
        pltpu.sync_copy(snorm, dgb.at[pl.ds(n0, nt)])
        _fill(ones, EB, 1, 1.0)
        plsc.subcore_barrier()

        pltpu.sync_copy(src_hbm.at[pl.ds(s * nb, nb)], srcg0)
        pltpu.sync_copy(dst_hbm.at[pl.ds(s * nb, nb)], dstv)

        def deg_body(j, _):
            da = pltpu.async_copy(ones, dga.at[srcg0.at[j]], ss0, add=True)
            db = pltpu.async_copy(ones, dgb.at[dstv.at[j]], ss1, add=True)
            da.wait()
            db.wait()
            return _

        lax.fori_loop(0, nb, deg_body, None)

        off0 = jnp.full((L,), 2 * c * n, jnp.int32)

        def _shift(delta):
            dv = jnp.full((L,), delta, jnp.int32)

            def body(j, _):
                for v in range(EB // L):
                    sl = pl.ds(v * L, L)
                    srcg0[j, sl] = srcg0[j, sl] + dv
                return _

            lax.fori_loop(0, nb, body, None)

        def off_body(j, _):
            for v in range(EB // L):
                sl = pl.ds(v * L, L)
                srcg0[j, sl] = srcg0[j, sl] + off0
            return _

        lax.fori_loop(0, nb, off_body, None)
        plsc.subcore_barrier()

        pltpu.sync_copy(dga.at[pl.ds(n0, nt)], snorm)
        pltpu.sync_copy(dgb.at[pl.ds(n0, nt)], dnorm)

        def norm_body(r, _):
            sl = pl.ds(0, L)
            snorm[r, sl] = _rsqrt16(jnp.maximum(snorm[r, sl], 1.0))
            dnorm[r, sl] = _rsqrt16(jnp.maximum(dnorm[r, sl], 1.0))
            return _

        lax.fori_loop(0, nt, norm_body, None)

        for p in range(2):
            yq = (2 * c + p) * n + n0
            for t in range(nch):
                pltpu.sync_copy(y0_hbm.at[pl.ds(yq + t * ch, ch)], ych)

                def h0_body(r, _):
                    sn = snorm[t * ch + r, pl.ds(0, L)]
                    for v in range(vq):
                        sl = pl.ds(v * L, L)
                        ych[r, sl] = ych[r, sl] * sn
                    return _

                lax.fori_loop(0, ch, h0_body, None)
                pltpu.sync_copy(ych, h_hbm.at[pl.ds(yq + t * ch, ch)])
        plsc.subcore_barrier()

        for k in range(K_STEPS):
            last = k == K_STEPS - 1
            for p in range(2):
                if p == 1:
                    _shift(n)

                def gather(j, buf, sem):
                    return pltpu.async_copy(h_hbm.at[srcg0.at[j]], buf, sem)

                def scat(j, buf, sem):
                    return pltpu.async_copy(
                        buf, uacc.at[dstv.at[j]], sem, add=True)

                gather(0, gbuf0, gs0)
                gather(1, gbuf1, gs1)

                def edge_pair(i, _):
                    j = 2 * i
                    pltpu.make_async_copy(h_hbm.at[srcg0.at[j]],
                                          gbuf0, gs0).wait()
                    sd0 = scat(j, gbuf0, ss0)
                    pltpu.make_async_copy(h_hbm.at[srcg0.at[j + 1]],
                                          gbuf1, gs1).wait()
                    sd0.wait()
                    gather(j + 2, gbuf0, gs0)
                    sd1 = scat(j + 1, gbuf1, ss1)
                    sd1.wait()

                    @pl.when(j + 3 < nb)
                    def _():
                        gather(j + 3, gbuf1, gs1)

                    return _

                lax.fori_loop(0, (nb - 1) // 2, edge_pair, None)
                pltpu.make_async_copy(h_hbm.at[srcg0.at[nb - 1]],
                                      gbuf0, gs0).wait()
                scat(nb - 1, gbuf0, ss0).wait()
                if p == 1:
                    _shift(-n)
                plsc.subcore_barrier()

                for t in range(nch):
                    g0 = n0 + t * ch
                    yq = (2 * c + p) * n + g0
                    pltpu.sync_copy(uacc.at[pl.ds(g0, ch)], uch)
                    pltpu.sync_copy(y0_hbm.at[pl.ds(yq, ch)], ych)

                    def dense_body(r, _):
                        sn = snorm[t * ch + r, pl.ds(0, L)]
                        dn = dnorm[t * ch + r, pl.ds(0, L)]
                        if last:
                            a = (1.0 - ALPHA) * dn
                            b = jnp.full((L,), ALPHA, jnp.float32)
                        else:
                            a = (1.0 - ALPHA) * sn * dn
                            b = ALPHA * sn
                        for v in range(vq):
                            sl = pl.ds(v * L, L)
                            uch[r, sl] = a * uch[r, sl] + b * ych[r, sl]
                        return _

                    lax.fori_loop(0, ch, dense_body, None)
                    if last:
                        pltpu.sync_copy(
                            uch,
                            out_hbm.at[pl.ds(g0, ch),
                                       pl.ds((2 * c + p) * DQ, DQ)])
                    else:
                        pltpu.sync_copy(uch, h_hbm.at[pl.ds(yq, ch)])
                    if not (last and p == 1):
                        _fill(ych, ch, vq, 0.0)
                        pltpu.sync_copy(ych, uacc.at[pl.ds(g0, ch)])
                if not (last and p == 1):
                    plsc.subcore_barrier()

    return prop


def _tc_matmul(n, d, bn):
    def body(x_ref, w_ref, o_ref):
        o_ref[...] = lax.dot_general(
            x_ref[...], w_ref[...], (((1,), (1,)), ((), ())),
            preferred_element_type=jnp.float32)

    nblk = n // bn
    return pl.pallas_call(
        body,
        grid=(4, nblk),
        in_specs=[
            pl.BlockSpec((bn, d), lambda q, i: (i, 0)),
            pl.BlockSpec((DQ, d), lambda q, i: (q, 0)),
        ],
        out_specs=pl.BlockSpec((bn, DQ), lambda q, i: (q * nblk + i, 0)),
        out_shape=jax.ShapeDtypeStruct((4 * n, DQ), jnp.float32),
    )


def kernel(x, edge_index, W):
    n, d = x.shape
    e = edge_index.shape[1]

    src = edge_index[0].astype(jnp.int32)
    dst = edge_index[1].astype(jnp.int32)
    nb = e // (NS * EB)
    src2 = src.reshape(NS * nb, EB)
    dst2 = dst.reshape(NS * nb, EB)

    nt = n // NS
    ch = 125
    nch = nt // ch

    y0 = _tc_matmul(n, d, bn=400)(x, W)
    out, _ = _sc_propagate(n, e, nb, nt, nch, ch)(y0, src2, dst2)
    return out


# speedup vs baseline: 4.6526x; 1.1109x over previous
import functools

import jax
import jax.numpy as jnp
from jax import lax
from jax.experimental import pallas as pl
from jax.experimental.pallas import tpu as pltpu
from jax.experimental.pallas import tpu_sc as plsc

K_STEPS = 3
ALPHA = 0.5
NS = 16
NC = 2
L = 16
EB = 80
DQ = 64


def _rsqrt16(d):
    half = d * 0.5
    i = plsc.bitcast(d, jnp.int32)
    i = jnp.full((L,), 0x5F3759DF, jnp.int32) - lax.shift_right_arithmetic(
        i, jnp.full((L,), 1, jnp.int32))
    y = plsc.bitcast(i, jnp.float32)
    for _ in range(3):
        y = y * (1.5 - half * y * y)
    return y


def _fill(ref, rows, vec16s, value):
    v = jnp.full((L,), value, jnp.float32)

    @plsc.parallel_loop(0, rows, unroll=4)
    def body(r):
        for j in range(vec16s):
            ref[r, pl.ds(j * L, L)] = v


def _sc_propagate(n, e, nb, nt, nch, ch):
    vq = DQ // L

    mesh = plsc.VectorSubcoreMesh(core_axis_name="c", subcore_axis_name="s")

    @functools.partial(
        pl.kernel,
        mesh=mesh,
        compiler_params=pltpu.CompilerParams(
            use_tc_tiling_on_sc=False, needs_layout_passes=False),
        out_type=(
            jax.ShapeDtypeStruct((n, 4 * DQ), jnp.float32),
            jax.ShapeDtypeStruct((4 * n, DQ), jnp.float32),
        ),
        scratch_types=dict(
            uacc=pltpu.VMEM_SHARED((n, DQ), jnp.float32),
            dga=pltpu.VMEM_SHARED((n, L), jnp.float32),
            dgb=pltpu.VMEM_SHARED((n, L), jnp.float32),
            srcg0=pltpu.VMEM((nb, EB), jnp.int32),
            dstv=pltpu.VMEM((nb, EB), jnp.int32),
            gbuf0=pltpu.VMEM((EB, DQ), jnp.float32),
            gbuf1=pltpu.VMEM((EB, DQ), jnp.float32),
            ones=pltpu.VMEM((EB, L), jnp.float32),
            snorm=pltpu.VMEM((nt, L), jnp.float32),
            dnorm=pltpu.VMEM((nt, L), jnp.float32),
            uch=pltpu.VMEM((ch, DQ), jnp.float32),
            ych=pltpu.VMEM((ch, DQ), jnp.float32),
            gs0=pltpu.SemaphoreType.DMA,
            gs1=pltpu.SemaphoreType.DMA,
            ss0=pltpu.SemaphoreType.DMA,
            ss1=pltpu.SemaphoreType.DMA,
        ),
    )
    def prop(y0_hbm, src_hbm, dst_hbm, out_hbm, h_hbm,
             uacc, dga, dgb, srcg0, dstv, gbuf0, gbuf1, ones, snorm, dnorm,
             uch, ych, gs0, gs1, ss0, ss1):
        c = lax.axis_index("c")
        s = lax.axis_index("s")
        n0 = s * nt

        _fill(ych, ch, vq, 0.0)
        for t in range(nch):
            pltpu.sync_copy(ych, uacc.at[pl.ds(n0 + t * ch, ch)])
        _fill(snorm, nt, 1, 0.0)
        pltpu.sync_copy(snorm, dga.at[pl.ds(n0, nt)])
        pltpu.sync_copy(snorm, dgb.at[pl.ds(n0, nt)])
        _fill(ones, EB, 1, 1.0)
        plsc.subcore_barrier()

        pltpu.sync_copy(src_hbm.at[pl.ds(s * nb, nb)], srcg0)
        pltpu.sync_copy(dst_hbm.at[pl.ds(s * nb, nb)], dstv)

        lag = 4

        def deg_wait():
            pltpu.make_async_copy(ones, dga.at[srcg0.at[0]], ss0).wait()
            pltpu.make_async_copy(ones, dgb.at[dstv.at[0]], ss1).wait()

        def deg_body(j, _):
            pltpu.async_copy(ones, dga.at[srcg0.at[j]], ss0, add=True)
            pltpu.async_copy(ones, dgb.at[dstv.at[j]], ss1, add=True)

            @pl.when(j >= lag)
            def _w():
                deg_wait()

            return _

        lax.fori_loop(0, nb, deg_body, None)
        for _ in range(lag):
            deg_wait()

        def _shift(delta):
            dv = jnp.full((L,), delta, jnp.int32)

            @plsc.parallel_loop(0, nb, unroll=4)
            def body(j):
                for v in range(EB // L):
                    sl = pl.ds(v * L, L)
                    srcg0[j, sl] = srcg0[j, sl] + dv

        _shift(2 * c * n)
        plsc.subcore_barrier()

        pltpu.sync_copy(dga.at[pl.ds(n0, nt)], snorm)
        pltpu.sync_copy(dgb.at[pl.ds(n0, nt)], dnorm)

        @plsc.parallel_loop(0, nt, unroll=2)
        def norm_body(r):
            sl = pl.ds(0, L)
            snorm[r, sl] = _rsqrt16(jnp.maximum(snorm[r, sl], 1.0))
            dnorm[r, sl] = _rsqrt16(jnp.maximum(dnorm[r, sl], 1.0))

        for p in range(2):
            yq = (2 * c + p) * n + n0
            for t in range(nch):
                pltpu.sync_copy(y0_hbm.at[pl.ds(yq + t * ch, ch)], ych)

                @plsc.parallel_loop(0, ch, unroll=4)
                def h0_body(r):
                    sn = snorm[t * ch + r, pl.ds(0, L)]
                    for v in range(vq):
                        sl = pl.ds(v * L, L)
                        ych[r, sl] = ych[r, sl] * sn
                pltpu.sync_copy(ych, h_hbm.at[pl.ds(yq + t * ch, ch)])
        plsc.subcore_barrier()

        for k in range(K_STEPS):
            last = k == K_STEPS - 1
            for p in range(2):
                if p == 1:
                    _shift(n)

                def gather(j, buf, sem):
                    return pltpu.async_copy(h_hbm.at[srcg0.at[j]], buf, sem)

                def scat(j, buf, sem):
                    return pltpu.async_copy(
                        buf, uacc.at[dstv.at[j]], sem, add=True)

                gather(0, gbuf0, gs0)
                gather(1, gbuf1, gs1)

                def edge_pair(i, _):
                    j = 2 * i
                    pltpu.make_async_copy(h_hbm.at[srcg0.at[j]],
                                          gbuf0, gs0).wait()
                    sd0 = scat(j, gbuf0, ss0)
                    pltpu.make_async_copy(h_hbm.at[srcg0.at[j + 1]],
                                          gbuf1, gs1).wait()
                    sd0.wait()
                    gather(j + 2, gbuf0, gs0)
                    sd1 = scat(j + 1, gbuf1, ss1)
                    sd1.wait()

                    @pl.when(j + 3 < nb)
                    def _():
                        gather(j + 3, gbuf1, gs1)

                    return _

                lax.fori_loop(0, (nb - 1) // 2, edge_pair, None)
                pltpu.make_async_copy(h_hbm.at[srcg0.at[nb - 1]],
                                      gbuf0, gs0).wait()
                scat(nb - 1, gbuf0, ss0).wait()
                if p == 1:
                    _shift(-n)
                plsc.subcore_barrier()

                for t in range(nch):
                    g0 = n0 + t * ch
                    yq = (2 * c + p) * n + g0
                    pltpu.sync_copy(uacc.at[pl.ds(g0, ch)], uch)
                    pltpu.sync_copy(y0_hbm.at[pl.ds(yq, ch)], ych)

                    @plsc.parallel_loop(0, ch, unroll=4)
                    def dense_body(r):
                        sn = snorm[t * ch + r, pl.ds(0, L)]
                        dn = dnorm[t * ch + r, pl.ds(0, L)]
                        if last:
                            a = (1.0 - ALPHA) * dn
                            b = jnp.full((L,), ALPHA, jnp.float32)
                        else:
                            a = (1.0 - ALPHA) * sn * dn
                            b = ALPHA * sn
                        for v in range(vq):
                            sl = pl.ds(v * L, L)
                            uch[r, sl] = a * uch[r, sl] + b * ych[r, sl]
                    if last:
                        pltpu.sync_copy(
                            uch,
                            out_hbm.at[pl.ds(g0, ch),
                                       pl.ds((2 * c + p) * DQ, DQ)])
                    else:
                        pltpu.sync_copy(uch, h_hbm.at[pl.ds(yq, ch)])
                    if not (last and p == 1):
                        _fill(ych, ch, vq, 0.0)
                        pltpu.sync_copy(ych, uacc.at[pl.ds(g0, ch)])
                if not (last and p == 1):
                    plsc.subcore_barrier()

    return prop


def _tc_matmul(n, d, bn):
    def body(x_ref, w_ref, o_ref):
        o_ref[...] = lax.dot_general(
            x_ref[...], w_ref[...], (((1,), (1,)), ((), ())),
            preferred_element_type=jnp.float32)

    nblk = n // bn
    return pl.pallas_call(
        body,
        grid=(4, nblk),
        in_specs=[
            pl.BlockSpec((bn, d), lambda q, i: (i, 0)),
            pl.BlockSpec((DQ, d), lambda q, i: (q, 0)),
        ],
        out_specs=pl.BlockSpec((bn, DQ), lambda q, i: (q * nblk + i, 0)),
        out_shape=jax.ShapeDtypeStruct((4 * n, DQ), jnp.float32),
    )


def kernel(x, edge_index, W):
    n, d = x.shape
    e = edge_index.shape[1]

    src = edge_index[0].astype(jnp.int32)
    dst = edge_index[1].astype(jnp.int32)
    nb = e // (NS * EB)
    src2 = src.reshape(NS * nb, EB)
    dst2 = dst.reshape(NS * nb, EB)

    nt = n // NS
    ch = 125
    nch = nt // ch

    y0 = _tc_matmul(n, d, bn=400)(x, W)
    out, _ = _sc_propagate(n, e, nb, nt, nch, ch)(y0, src2, dst2)
    return out


# speedup vs baseline: 5.1288x; 1.1024x over previous
import functools

import jax
import jax.numpy as jnp
from jax import lax
from jax.experimental import pallas as pl
from jax.experimental.pallas import tpu as pltpu
from jax.experimental.pallas import tpu_sc as plsc

K_STEPS = 3
ALPHA = 0.5
NS = 16
NC = 2
L = 16
EB = 80
DQ = 64


def _rsqrt16(d):
    half = d * 0.5
    i = plsc.bitcast(d, jnp.int32)
    i = jnp.full((L,), 0x5F3759DF, jnp.int32) - lax.shift_right_arithmetic(
        i, jnp.full((L,), 1, jnp.int32))
    y = plsc.bitcast(i, jnp.float32)
    for _ in range(3):
        y = y * (1.5 - half * y * y)
    return y


def _fill(ref, rows, vec16s, value):
    v = jnp.full((L,), value, jnp.float32)

    @plsc.parallel_loop(0, rows, unroll=4)
    def body(r):
        for j in range(vec16s):
            ref[r, pl.ds(j * L, L)] = v


def _sc_propagate(n, e, nb, nt, nch, ch):
    vq = DQ // L

    mesh = plsc.VectorSubcoreMesh(core_axis_name="c", subcore_axis_name="s")

    @functools.partial(
        pl.kernel,
        mesh=mesh,
        compiler_params=pltpu.CompilerParams(
            use_tc_tiling_on_sc=False, needs_layout_passes=False),
        out_type=(
            jax.ShapeDtypeStruct((n, 4 * DQ), jnp.float32),
            jax.ShapeDtypeStruct((4 * n, DQ), jnp.float32),
        ),
        scratch_types=dict(
            uacc=pltpu.VMEM_SHARED((n, DQ), jnp.float32),
            dga=pltpu.VMEM_SHARED((n, L), jnp.float32),
            srcg0=pltpu.VMEM((nb, EB), jnp.int32),
            dstv=pltpu.VMEM((nb, EB), jnp.int32),
            gbuf0=pltpu.VMEM((EB, DQ), jnp.float32),
            gbuf1=pltpu.VMEM((EB, DQ), jnp.float32),
            gbuf2=pltpu.VMEM((EB, DQ), jnp.float32),
            ones=pltpu.VMEM((EB, L), jnp.float32),
            snorm=pltpu.VMEM((nt, L), jnp.float32),
            dnorm=pltpu.VMEM((nt, L), jnp.float32),
            uch=pltpu.VMEM((ch, DQ), jnp.float32),
            ych=pltpu.VMEM((ch, DQ), jnp.float32),
            gs0=pltpu.SemaphoreType.DMA,
            gs1=pltpu.SemaphoreType.DMA,
            gs2=pltpu.SemaphoreType.DMA,
            ss0=pltpu.SemaphoreType.DMA,
            ss1=pltpu.SemaphoreType.DMA,
            ss2=pltpu.SemaphoreType.DMA,
        ),
    )
    def prop(y0_hbm, src_hbm, dst_hbm, out_hbm, h_hbm,
             uacc, dga, srcg0, dstv, gbuf0, gbuf1, gbuf2, ones, snorm, dnorm,
             uch, ych, gs0, gs1, gs2, ss0, ss1, ss2):
        c = lax.axis_index("c")
        s = lax.axis_index("s")
        n0 = s * nt

        _fill(ych, ch, vq, 0.0)
        for t in range(nch):
            pltpu.sync_copy(ych, uacc.at[pl.ds(n0 + t * ch, ch)])
        _fill(snorm, nt, 1, 0.0)
        pltpu.sync_copy(snorm, dga.at[pl.ds(n0, nt)])
        _fill(ones, EB, 1, 1.0)
        plsc.subcore_barrier()

        pltpu.sync_copy(src_hbm.at[pl.ds(s * nb, nb)], srcg0)
        pltpu.sync_copy(dst_hbm.at[pl.ds(s * nb, nb)], dstv)
        lag = 6

        def deg_scatter(idx):
            def deg_wait():
                pltpu.make_async_copy(ones, dga.at[idx.at[0]], ss0).wait()

            def deg_body(j, _):
                pltpu.async_copy(ones, dga.at[idx.at[j]], ss0, add=True)

                @pl.when(j >= lag)
                def _w():
                    deg_wait()

                return _

            lax.fori_loop(0, nb, deg_body, None)
            for _ in range(lag):
                deg_wait()

        def _shift(delta):
            dv = jnp.full((L,), delta, jnp.int32)

            @plsc.parallel_loop(0, nb, unroll=4)
            def body(j):
                for v in range(EB // L):
                    sl = pl.ds(v * L, L)
                    srcg0[j, sl] = srcg0[j, sl] + dv

        deg_scatter(srcg0)
        _shift(2 * c * n)
        plsc.subcore_barrier()

        pltpu.sync_copy(dga.at[pl.ds(n0, nt)], snorm)
        _fill(dnorm, nt, 1, 0.0)
        pltpu.sync_copy(dnorm, dga.at[pl.ds(n0, nt)])
        plsc.subcore_barrier()
        deg_scatter(dstv)
        plsc.subcore_barrier()
        pltpu.sync_copy(dga.at[pl.ds(n0, nt)], dnorm)

        @plsc.parallel_loop(0, nt, unroll=2)
        def norm_body(r):
            sl = pl.ds(0, L)
            snorm[r, sl] = _rsqrt16(jnp.maximum(snorm[r, sl], 1.0))
            dnorm[r, sl] = _rsqrt16(jnp.maximum(dnorm[r, sl], 1.0))

        for p in range(2):
            yq = (2 * c + p) * n + n0
            for t in range(nch):
                pltpu.sync_copy(y0_hbm.at[pl.ds(yq + t * ch, ch)], ych)

                @plsc.parallel_loop(0, ch, unroll=4)
                def h0_body(r):
                    sn = snorm[t * ch + r, pl.ds(0, L)]
                    for v in range(vq):
                        sl = pl.ds(v * L, L)
                        ych[r, sl] = ych[r, sl] * sn
                pltpu.sync_copy(ych, h_hbm.at[pl.ds(yq + t * ch, ch)])
        plsc.subcore_barrier()

        for k in range(K_STEPS):
            last = k == K_STEPS - 1
            for p in range(2):
                if p == 1:
                    _shift(n)

                bufs = (gbuf0, gbuf1, gbuf2)
                gss = (gs0, gs1, gs2)
                sss = (ss0, ss1, ss2)

                def gather(j, q):
                    pltpu.async_copy(h_hbm.at[srcg0.at[j]], bufs[q], gss[q])

                def gwait(q):
                    pltpu.make_async_copy(
                        h_hbm.at[srcg0.at[0]], bufs[q], gss[q]).wait()

                def scat(j, q):
                    pltpu.async_copy(
                        bufs[q], uacc.at[dstv.at[j]], sss[q], add=True)

                def swait(q):
                    pltpu.make_async_copy(
                        bufs[q], uacc.at[dstv.at[0]], sss[q]).wait()

                for q in range(3):
                    gather(q, q)

                def edge_tri(i, _):
                    j = 3 * i
                    for q in range(3):
                        gwait(q)
                        scat(j + q, q)
                    for q in range(3):
                        swait(q)

                        @pl.when(j + 3 + q < nb)
                        def _g(jq=j + 3 + q, q=q):
                            gather(jq, q)

                    return _

                lax.fori_loop(0, nb // 3, edge_tri, None)
                for q in range(nb - 3 * (nb // 3)):
                    gwait(q)
                    scat(3 * (nb // 3) + q, q)
                    swait(q)
                if p == 1:
                    _shift(-n)
                plsc.subcore_barrier()

                for t in range(nch):
                    g0 = n0 + t * ch
                    yq = (2 * c + p) * n + g0
                    pltpu.sync_copy(uacc.at[pl.ds(g0, ch)], uch)
                    pltpu.sync_copy(y0_hbm.at[pl.ds(yq, ch)], ych)

                    @plsc.parallel_loop(0, ch, unroll=4)
                    def dense_body(r):
                        sn = snorm[t * ch + r, pl.ds(0, L)]
                        dn = dnorm[t * ch + r, pl.ds(0, L)]
                        if last:
                            a = (1.0 - ALPHA) * dn
                            b = jnp.full((L,), ALPHA, jnp.float32)
                        else:
                            a = (1.0 - ALPHA) * sn * dn
                            b = ALPHA * sn
                        for v in range(vq):
                            sl = pl.ds(v * L, L)
                            uch[r, sl] = a * uch[r, sl] + b * ych[r, sl]
                    if last:
                        pltpu.sync_copy(
                            uch,
                            out_hbm.at[pl.ds(g0, ch),
                                       pl.ds((2 * c + p) * DQ, DQ)])
                    else:
                        pltpu.sync_copy(uch, h_hbm.at[pl.ds(yq, ch)])
                    if not (last and p == 1):
                        _fill(ych, ch, vq, 0.0)
                        pltpu.sync_copy(ych, uacc.at[pl.ds(g0, ch)])
                if not (last and p == 1):
                    plsc.subcore_barrier()

    return prop


def _tc_matmul(n, d, bn):
    def body(x_ref, w_ref, o_ref):
        o_ref[...] = lax.dot_general(
            x_ref[...], w_ref[...], (((1,), (1,)), ((), ())),
            preferred_element_type=jnp.float32)

    nblk = n // bn
    return pl.pallas_call(
        body,
        grid=(4, nblk),
        in_specs=[
            pl.BlockSpec((bn, d), lambda q, i: (i, 0)),
            pl.BlockSpec((DQ, d), lambda q, i: (q, 0)),
        ],
        out_specs=pl.BlockSpec((bn, DQ), lambda q, i: (q * nblk + i, 0)),
        out_shape=jax.ShapeDtypeStruct((4 * n, DQ), jnp.float32),
    )


def kernel(x, edge_index, W):
    n, d = x.shape
    e = edge_index.shape[1]

    src = edge_index[0].astype(jnp.int32)
    dst = edge_index[1].astype(jnp.int32)
    nb = e // (NS * EB)
    src2 = src.reshape(NS * nb, EB)
    dst2 = dst.reshape(NS * nb, EB)

    nt = n // NS
    ch = 125
    nch = nt // ch

    y0 = _tc_matmul(n, d, bn=400)(x, W)
    out, _ = _sc_propagate(n, e, nb, nt, nch, ch)(y0, src2, dst2)
    return out


# speedup vs baseline: 5.5224x; 1.0767x over previous
import functools

import jax
import jax.numpy as jnp
from jax import lax
from jax.experimental import pallas as pl
from jax.experimental.pallas import tpu as pltpu
from jax.experimental.pallas import tpu_sc as plsc

K_STEPS = 3
ALPHA = 0.5
NS = 16
NC = 2
L = 16
EB = 80
DQ = 64


def _rsqrt16(d):
    half = d * 0.5
    i = plsc.bitcast(d, jnp.int32)
    i = jnp.full((L,), 0x5F3759DF, jnp.int32) - lax.shift_right_arithmetic(
        i, jnp.full((L,), 1, jnp.int32))
    y = plsc.bitcast(i, jnp.float32)
    for _ in range(3):
        y = y * (1.5 - half * y * y)
    return y


def _fill(ref, rows, vec16s, value):
    v = jnp.full((L,), value, jnp.float32)

    @plsc.parallel_loop(0, rows, unroll=4)
    def body(r):
        for j in range(vec16s):
            ref[r, pl.ds(j * L, L)] = v


def _sc_propagate(n, e, nb, nt, nch, ch):
    vq = DQ // L

    mesh = plsc.VectorSubcoreMesh(core_axis_name="c", subcore_axis_name="s")

    @functools.partial(
        pl.kernel,
        mesh=mesh,
        compiler_params=pltpu.CompilerParams(
            use_tc_tiling_on_sc=False, needs_layout_passes=False),
        out_type=(
            jax.ShapeDtypeStruct((n, 4 * DQ), jnp.float32),
            jax.ShapeDtypeStruct((4 * n, DQ), jnp.float32),
        ),
        scratch_types=dict(
            uacc=pltpu.VMEM_SHARED((n, DQ), jnp.float32),
            dga=pltpu.VMEM_SHARED((n, L), jnp.float32),
            srcg0=pltpu.VMEM((nb, EB), jnp.int32),
            dstv=pltpu.VMEM((nb, EB), jnp.int32),
            gbuf0=pltpu.VMEM((EB, DQ), jnp.float32),
            gbuf1=pltpu.VMEM((EB, DQ), jnp.float32),
            gbuf2=pltpu.VMEM((EB, DQ), jnp.float32),
            ones=pltpu.VMEM((EB, L), jnp.float32),
            snorm=pltpu.VMEM((nt, L), jnp.float32),
            dnorm=pltpu.VMEM((nt, L), jnp.float32),
            uch=pltpu.VMEM((ch, DQ), jnp.float32),
            ych=pltpu.VMEM((ch, DQ), jnp.float32),
            ych2=pltpu.VMEM((ch, DQ), jnp.float32),
            gs0=pltpu.SemaphoreType.DMA,
            gs1=pltpu.SemaphoreType.DMA,
            gs2=pltpu.SemaphoreType.DMA,
            ss0=pltpu.SemaphoreType.DMA,
            ss1=pltpu.SemaphoreType.DMA,
            ss2=pltpu.SemaphoreType.DMA,
        ),
    )
    def prop(y0_hbm, src_hbm, dst_hbm, out_hbm, h_hbm,
             uacc, dga, srcg0, dstv, gbuf0, gbuf1, gbuf2, ones, snorm, dnorm,
             uch, ych, ych2, gs0, gs1, gs2, ss0, ss1, ss2):
        c = lax.axis_index("c")
        s = lax.axis_index("s")
        n0 = s * nt

        _fill(ych, ch, vq, 0.0)
        for t in range(nch):
            pltpu.sync_copy(ych, uacc.at[pl.ds(n0 + t * ch, ch)])
        _fill(snorm, nt, 1, 0.0)
        pltpu.sync_copy(snorm, dga.at[pl.ds(n0, nt)])
        _fill(ones, EB, 1, 1.0)
        plsc.subcore_barrier()

        pltpu.sync_copy(src_hbm.at[pl.ds(s * nb, nb)], srcg0)
        pltpu.sync_copy(dst_hbm.at[pl.ds(s * nb, nb)], dstv)
        lag = 6

        def deg_scatter(idx):
            def deg_wait():
                pltpu.make_async_copy(ones, dga.at[idx.at[0]], ss0).wait()

            def deg_body(j, _):
                pltpu.async_copy(ones, dga.at[idx.at[j]], ss0, add=True)

                @pl.when(j >= lag)
                def _w():
                    deg_wait()

                return _

            lax.fori_loop(0, nb, deg_body, None)
            for _ in range(lag):
                deg_wait()

        def _shift(delta):
            dv = jnp.full((L,), delta, jnp.int32)

            @plsc.parallel_loop(0, nb, unroll=4)
            def body(j):
                for v in range(EB // L):
                    sl = pl.ds(v * L, L)
                    srcg0[j, sl] = srcg0[j, sl] + dv

        deg_scatter(srcg0)
        _shift(2 * c * n)
        plsc.subcore_barrier()

        pltpu.sync_copy(dga.at[pl.ds(n0, nt)], snorm)
        _fill(dnorm, nt, 1, 0.0)
        pltpu.sync_copy(dnorm, dga.at[pl.ds(n0, nt)])
        plsc.subcore_barrier()
        deg_scatter(dstv)
        plsc.subcore_barrier()
        pltpu.sync_copy(dga.at[pl.ds(n0, nt)], dnorm)

        @plsc.parallel_loop(0, nt, unroll=2)
        def norm_body(r):
            sl = pl.ds(0, L)
            snorm[r, sl] = _rsqrt16(jnp.maximum(snorm[r, sl], 1.0))
            dnorm[r, sl] = _rsqrt16(jnp.maximum(dnorm[r, sl], 1.0))

        ybufs = (ych, ych2)
        ysem = (gs0, gs1)
        stsem = (ss1, ss2)

        def yq_of(i):
            return (2 * c + (i // nch)) * n + n0 + (i % nch) * ch

        pend_st = [None, None]
        pend_y = [None, None]
        pend_y[0] = pltpu.async_copy(
            y0_hbm.at[pl.ds(yq_of(0), ch)], ybufs[0], ysem[0])
        for i in range(2 * nch):
            qq = i % 2
            yb = ybufs[qq]
            ti = i % nch
            pend_y[qq].wait()
            if i + 1 < 2 * nch:
                q2 = (i + 1) % 2
                if pend_st[q2] is not None:
                    pend_st[q2].wait()
                pend_y[q2] = pltpu.async_copy(
                    y0_hbm.at[pl.ds(yq_of(i + 1), ch)], ybufs[q2], ysem[q2])

            @plsc.parallel_loop(0, ch, unroll=4)
            def h0_body(r):
                sn = snorm[ti * ch + r, pl.ds(0, L)]
                for v in range(vq):
                    sl = pl.ds(v * L, L)
                    yb[r, sl] = yb[r, sl] * sn

            pend_st[qq] = pltpu.async_copy(
                yb, h_hbm.at[pl.ds(yq_of(i), ch)], stsem[qq])
        for qq in range(2):
            if pend_st[qq] is not None:
                pend_st[qq].wait()
        plsc.subcore_barrier()

        for k in range(K_STEPS):
            last = k == K_STEPS - 1
            for p in range(2):
                if p == 1:
                    _shift(n)

                bufs = (gbuf0, gbuf1, gbuf2)
                gss = (gs0, gs1, gs2)
                sss = (ss0, ss1, ss2)

                def gather(j, q):
                    pltpu.async_copy(h_hbm.at[srcg0.at[j]], bufs[q], gss[q])

                def gwait(q):
                    pltpu.make_async_copy(
                        h_hbm.at[srcg0.at[0]], bufs[q], gss[q]).wait()

                def scat(j, q):
                    pltpu.async_copy(
                        bufs[q], uacc.at[dstv.at[j]], sss[q], add=True)

                def swait(q):
                    pltpu.make_async_copy(
                        bufs[q], uacc.at[dstv.at[0]], sss[q]).wait()

                for q in range(3):
                    gather(q, q)

                def edge_tri(i, _):
                    j = 3 * i
                    for q in range(3):
                        gwait(q)
                        scat(j + q, q)
                    for q in range(3):
                        swait(q)

                        @pl.when(j + 3 + q < nb)
                        def _g(jq=j + 3 + q, q=q):
                            gather(jq, q)

                    return _

                lax.fori_loop(0, nb // 3, edge_tri, None)
                for q in range(nb - 3 * (nb // 3)):
                    gwait(q)
                    scat(3 * (nb // 3) + q, q)
                    swait(q)
                if p == 1:
                    _shift(-n)
                plsc.subcore_barrier()

                yq0 = (2 * c + p) * n + n0
                pend_h = None
                pend_z = [None, None]
                pend_yd = [None, None]
                pend_yd[0] = pltpu.async_copy(
                    y0_hbm.at[pl.ds(yq0, ch)], ybufs[0], ysem[0])
                for t in range(nch):
                    g0 = n0 + t * ch
                    if pend_h is not None:
                        pend_h.wait()
                    pltpu.sync_copy(uacc.at[pl.ds(g0, ch)], uch)
                    pend_yd[t % 2].wait()
                    if t + 1 < nch:
                        q2 = (t + 1) % 2
                        if pend_z[q2] is not None:
                            pend_z[q2].wait()
                            pend_z[q2] = None
                        pend_yd[q2] = pltpu.async_copy(
                            y0_hbm.at[pl.ds(yq0 + (t + 1) * ch, ch)],
                            ybufs[q2], ysem[q2])
                    yb = ybufs[t % 2]

                    @plsc.parallel_loop(0, ch, unroll=4)
                    def dense_body(r):
                        sn = snorm[t * ch + r, pl.ds(0, L)]
                        dn = dnorm[t * ch + r, pl.ds(0, L)]
                        if last:
                            a = (1.0 - ALPHA) * dn
                            b = jnp.full((L,), ALPHA, jnp.float32)
                        else:
                            a = (1.0 - ALPHA) * sn * dn
                            b = ALPHA * sn
                        for v in range(vq):
                            sl = pl.ds(v * L, L)
                            uch[r, sl] = a * uch[r, sl] + b * yb[r, sl]
                    if last:
                        pend_h = pltpu.async_copy(
                            uch,
                            out_hbm.at[pl.ds(g0, ch),
                                       pl.ds((2 * c + p) * DQ, DQ)], ss0)
                    else:
                        pend_h = pltpu.async_copy(
                            uch, h_hbm.at[pl.ds(yq0 + t * ch, ch)], ss0)
                    if not (last and p == 1):
                        _fill(yb, ch, vq, 0.0)
                        pend_z[t % 2] = pltpu.async_copy(
                            yb, uacc.at[pl.ds(g0, ch)], stsem[t % 2])
                pend_h.wait()
                for q2 in range(2):
                    if pend_z[q2] is not None:
                        pend_z[q2].wait()
                if not (last and p == 1):
                    plsc.subcore_barrier()

    return prop


def _tc_matmul(n, d, bn):
    def body(x_ref, w_ref, o_ref):
        o_ref[...] = lax.dot_general(
            x_ref[...], w_ref[...], (((1,), (1,)), ((), ())),
            preferred_element_type=jnp.float32)

    nblk = n // bn
    return pl.pallas_call(
        body,
        grid=(4, nblk),
        in_specs=[
            pl.BlockSpec((bn, d), lambda q, i: (i, 0)),
            pl.BlockSpec((DQ, d), lambda q, i: (q, 0)),
        ],
        out_specs=pl.BlockSpec((bn, DQ), lambda q, i: (q * nblk + i, 0)),
        out_shape=jax.ShapeDtypeStruct((4 * n, DQ), jnp.float32),
    )


def kernel(x, edge_index, W):
    n, d = x.shape
    e = edge_index.shape[1]

    src = edge_index[0].astype(jnp.int32)
    dst = edge_index[1].astype(jnp.int32)
    nb = e // (NS * EB)
    src2 = src.reshape(NS * nb, EB)
    dst2 = dst.reshape(NS * nb, EB)

    nt = n // NS
    ch = 125
    nch = nt // ch

    y0 = _tc_matmul(n, d, bn=400)(x, W)
    out, _ = _sc_propagate(n, e, nb, nt, nch, ch)(y0, src2, dst2)
    return out


# speedup vs baseline: 6.1544x; 1.1144x over previous
import functools

import jax
import jax.numpy as jnp
from jax import lax
from jax.experimental import pallas as pl
from jax.experimental.pallas import tpu as pltpu
from jax.experimental.pallas import tpu_sc as plsc

K_STEPS = 3
ALPHA = 0.5
NS = 16
NC = 2
L = 16
EB = 80
DQ = 64


def _rsqrt16(d):
    half = d * 0.5
    i = plsc.bitcast(d, jnp.int32)
    i = jnp.full((L,), 0x5F3759DF, jnp.int32) - lax.shift_right_arithmetic(
        i, jnp.full((L,), 1, jnp.int32))
    y = plsc.bitcast(i, jnp.float32)
    for _ in range(3):
        y = y * (1.5 - half * y * y)
    return y


def _fill(ref, rows, vec16s, value):
    v = jnp.full((L,), value, jnp.float32)

    @plsc.parallel_loop(0, rows, unroll=4)
    def body(r):
        for j in range(vec16s):
            ref[r, pl.ds(j * L, L)] = v


def _sc_propagate(n, e, nb, nt, nch, ch):
    vq = DQ // L

    mesh = plsc.VectorSubcoreMesh(core_axis_name="c", subcore_axis_name="s")

    @functools.partial(
        pl.kernel,
        mesh=mesh,
        compiler_params=pltpu.CompilerParams(
            use_tc_tiling_on_sc=False, needs_layout_passes=False),
        out_type=(
            jax.ShapeDtypeStruct((n, 4 * DQ), jnp.float32),
            jax.ShapeDtypeStruct((4 * n, DQ), jnp.float32),
        ),
        scratch_types=dict(
            uacc=pltpu.VMEM_SHARED((n, DQ), jnp.float32),
            dga=pltpu.VMEM_SHARED((n, L), jnp.float32),
            srcg0=pltpu.VMEM((nb, EB), jnp.int32),
            dstv=pltpu.VMEM((nb, EB), jnp.int32),
            gbuf0=pltpu.VMEM((EB, DQ), jnp.float32),
            gbuf1=pltpu.VMEM((EB, DQ), jnp.float32),
            gbuf2=pltpu.VMEM((EB, DQ), jnp.float32),
            gbuf3=pltpu.VMEM((EB, DQ), jnp.float32),
            ones=pltpu.VMEM((EB, L), jnp.float32),
            nvec=pltpu.VMEM((nt, L), jnp.float32),
            snorm=pltpu.SMEM((nt,), jnp.float32),
            dnorm=pltpu.SMEM((nt,), jnp.float32),
            uch=pltpu.VMEM((ch, DQ), jnp.float32),
            ych=pltpu.VMEM((ch, DQ), jnp.float32),
            ych2=pltpu.VMEM((ch, DQ), jnp.float32),
            gs0=pltpu.SemaphoreType.DMA,
            gs1=pltpu.SemaphoreType.DMA,
            gs2=pltpu.SemaphoreType.DMA,
            gs3=pltpu.SemaphoreType.DMA,
            ss0=pltpu.SemaphoreType.DMA,
            ss1=pltpu.SemaphoreType.DMA,
            ss2=pltpu.SemaphoreType.DMA,
            ss3=pltpu.SemaphoreType.DMA,
        ),
    )
    def prop(y0_hbm, src_hbm, dst_hbm, out_hbm, h_hbm,
             uacc, dga, srcg0, dstv, gbuf0, gbuf1, gbuf2, gbuf3, ones, nvec,
             snorm, dnorm, uch, ych, ych2,
             gs0, gs1, gs2, gs3, ss0, ss1, ss2, ss3):
        c = lax.axis_index("c")
        s = lax.axis_index("s")
        n0 = s * nt

        _fill(ych, ch, vq, 0.0)
        for t in range(nch):
            pltpu.sync_copy(ych, uacc.at[pl.ds(n0 + t * ch, ch)])
        _fill(nvec, nt, 1, 0.0)
        pltpu.sync_copy(nvec, dga.at[pl.ds(n0, nt)])
        _fill(ones, EB, 1, 1.0)
        plsc.subcore_barrier()

        pltpu.sync_copy(src_hbm.at[pl.ds(s * nb, nb)], srcg0)
        pltpu.sync_copy(dst_hbm.at[pl.ds(s * nb, nb)], dstv)
        lag = 6

        def deg_scatter(idx):
            def deg_wait():
                pltpu.make_async_copy(ones, dga.at[idx.at[0]], ss0).wait()

            def deg_body(j, _):
                pltpu.async_copy(ones, dga.at[idx.at[j]], ss0, add=True)

                @pl.when(j >= lag)
                def _w():
                    deg_wait()

                return _

            lax.fori_loop(0, nb, deg_body, None)
            for _ in range(lag):
                deg_wait()

        def _shift(delta):
            dv = jnp.full((L,), delta, jnp.int32)

            @plsc.parallel_loop(0, nb, unroll=4)
            def body(j):
                for v in range(EB // L):
                    sl = pl.ds(v * L, L)
                    srcg0[j, sl] = srcg0[j, sl] + dv

        deg_scatter(srcg0)
        _shift(2 * c * n)
        plsc.subcore_barrier()

        def norms_to(dst1d):
            @plsc.parallel_loop(0, nt, unroll=2)
            def norm_body(r):
                y = _rsqrt16(jnp.maximum(nvec[r, pl.ds(0, L)], 1.0))
                dst1d[r] = y[0]

        pltpu.sync_copy(dga.at[pl.ds(n0, nt)], nvec)
        norms_to(snorm)
        _fill(nvec, nt, 1, 0.0)
        pltpu.sync_copy(nvec, dga.at[pl.ds(n0, nt)])
        plsc.subcore_barrier()
        deg_scatter(dstv)
        plsc.subcore_barrier()
        pltpu.sync_copy(dga.at[pl.ds(n0, nt)], nvec)
        norms_to(dnorm)

        ybufs = (ych, ych2)
        ysem = (gs0, gs1)
        stsem = (ss1, ss2)

        def yq_of(i):
            return (2 * c + (i // nch)) * n + n0 + (i % nch) * ch

        pend_st = [None, None]
        pend_y = [None, None]
        pend_y[0] = pltpu.async_copy(
            y0_hbm.at[pl.ds(yq_of(0), ch)], ybufs[0], ysem[0])
        for i in range(2 * nch):
            qq = i % 2
            yb = ybufs[qq]
            ti = i % nch
            pend_y[qq].wait()
            if i + 1 < 2 * nch:
                q2 = (i + 1) % 2
                if pend_st[q2] is not None:
                    pend_st[q2].wait()
                pend_y[q2] = pltpu.async_copy(
                    y0_hbm.at[pl.ds(yq_of(i + 1), ch)], ybufs[q2], ysem[q2])

            @plsc.parallel_loop(0, ch, unroll=4)
            def h0_body(r):
                sn = snorm[ti * ch + r]
                for v in range(vq):
                    sl = pl.ds(v * L, L)
                    yb[r, sl] = yb[r, sl] * sn

            pend_st[qq] = pltpu.async_copy(
                yb, h_hbm.at[pl.ds(yq_of(i), ch)], stsem[qq])
        for qq in range(2):
            if pend_st[qq] is not None:
                pend_st[qq].wait()
        plsc.subcore_barrier()

        for k in range(K_STEPS):
            last = k == K_STEPS - 1
            for p in range(2):
                if p == 1:
                    _shift(n)

                nbuf = 4
                bufs = (gbuf0, gbuf1, gbuf2, gbuf3)
                gss = (gs0, gs1, gs2, gs3)
                sss = (ss0, ss1, ss2, ss3)

                def gather(j, q):
                    pltpu.async_copy(h_hbm.at[srcg0.at[j]], bufs[q], gss[q])

                def gwait(q):
                    pltpu.make_async_copy(
                        h_hbm.at[srcg0.at[0]], bufs[q], gss[q]).wait()

                def scat(j, q):
                    pltpu.async_copy(
                        bufs[q], uacc.at[dstv.at[j]], sss[q], add=True)

                def swait(q):
                    pltpu.make_async_copy(
                        bufs[q], uacc.at[dstv.at[0]], sss[q]).wait()

                for q in range(nbuf):
                    gather(q, q)

                def edge_quad(i, _):
                    j = nbuf * i
                    for q in range(nbuf):
                        gwait(q)
                        scat(j + q, q)
                    for q in range(nbuf):
                        swait(q)

                        @pl.when(j + nbuf + q < nb)
                        def _g(jq=j + nbuf + q, q=q):
                            gather(jq, q)

                    return _

                lax.fori_loop(0, nb // nbuf, edge_quad, None)
                for q in range(nb - nbuf * (nb // nbuf)):
                    gwait(q)
                    scat(nbuf * (nb // nbuf) + q, q)
                    swait(q)
                if p == 1:
                    _shift(-n)
                plsc.subcore_barrier()

                yq0 = (2 * c + p) * n + n0
                pend_h = None
                pend_z = [None, None]
                pend_yd = [None, None]
                pend_yd[0] = pltpu.async_copy(
                    y0_hbm.at[pl.ds(yq0, ch)], ybufs[0], ysem[0])
                for t in range(nch):
                    g0 = n0 + t * ch
                    if pend_h is not None:
                        pend_h.wait()
                    pltpu.sync_copy(uacc.at[pl.ds(g0, ch)], uch)
                    pend_yd[t % 2].wait()
                    if t + 1 < nch:
                        q2 = (t + 1) % 2
                        if pend_z[q2] is not None:
                            pend_z[q2].wait()
                            pend_z[q2] = None
                        pend_yd[q2] = pltpu.async_copy(
                            y0_hbm.at[pl.ds(yq0 + (t + 1) * ch, ch)],
                            ybufs[q2], ysem[q2])
                    yb = ybufs[t % 2]

                    @plsc.parallel_loop(0, ch, unroll=4)
                    def dense_body(r):
                        sn = snorm[t * ch + r]
                        dn = dnorm[t * ch + r]
                        if last:
                            a = (1.0 - ALPHA) * dn
                            b = jnp.float32(ALPHA)
                        else:
                            a = (1.0 - ALPHA) * sn * dn
                            b = ALPHA * sn
                        for v in range(vq):
                            sl = pl.ds(v * L, L)
                            uch[r, sl] = a * uch[r, sl] + b * yb[r, sl]
                    if last:
                        pend_h = pltpu.async_copy(
                            uch,
                            out_hbm.at[pl.ds(g0, ch),
                                       pl.ds((2 * c + p) * DQ, DQ)], ss0)
                    else:
                        pend_h = pltpu.async_copy(
                            uch, h_hbm.at[pl.ds(yq0 + t * ch, ch)], ss0)
                    if not (last and p == 1):
                        _fill(yb, ch, vq, 0.0)
                        pend_z[t % 2] = pltpu.async_copy(
                            yb, uacc.at[pl.ds(g0, ch)], stsem[t % 2])
                pend_h.wait()
                for q2 in range(2):
                    if pend_z[q2] is not None:
                        pend_z[q2].wait()
                if not (last and p == 1):
                    plsc.subcore_barrier()

    return prop


def _tc_matmul(n, d, bn):
    def body(x_ref, w_ref, o_ref):
        o_ref[...] = lax.dot_general(
            x_ref[...], w_ref[...], (((1,), (1,)), ((), ())),
            preferred_element_type=jnp.float32)

    nblk = n // bn
    return pl.pallas_call(
        body,
        grid=(4, nblk),
        in_specs=[
            pl.BlockSpec((bn, d), lambda q, i: (i, 0)),
            pl.BlockSpec((DQ, d), lambda q, i: (q, 0)),
        ],
        out_specs=pl.BlockSpec((bn, DQ), lambda q, i: (q * nblk + i, 0)),
        out_shape=jax.ShapeDtypeStruct((4 * n, DQ), jnp.float32),
    )


def kernel(x, edge_index, W):
    n, d = x.shape
    e = edge_index.shape[1]

    src = edge_index[0].astype(jnp.int32)
    dst = edge_index[1].astype(jnp.int32)
    nb = e // (NS * EB)
    src2 = src.reshape(NS * nb, EB)
    dst2 = dst.reshape(NS * nb, EB)

    nt = n // NS
    ch = 125
    nch = nt // ch

    y0 = _tc_matmul(n, d, bn=400)(x, W)
    out, _ = _sc_propagate(n, e, nb, nt, nch, ch)(y0, src2, dst2)
    return out


# speedup vs baseline: 6.3069x; 1.0248x over previous
import functools

import jax
import jax.numpy as jnp
from jax import lax
from jax.experimental import pallas as pl
from jax.experimental.pallas import tpu as pltpu
from jax.experimental.pallas import tpu_sc as plsc

K_STEPS = 3
ALPHA = 0.5
NS = 16
NC = 2
L = 16
EB = 80
DQ = 64


def _rsqrt16(d):
    half = d * 0.5
    i = plsc.bitcast(d, jnp.int32)
    i = jnp.full((L,), 0x5F3759DF, jnp.int32) - lax.shift_right_arithmetic(
        i, jnp.full((L,), 1, jnp.int32))
    y = plsc.bitcast(i, jnp.float32)
    for _ in range(3):
        y = y * (1.5 - half * y * y)
    return y


def _fill(ref, rows, vec16s, value):
    v = jnp.full((L,), value, jnp.float32)

    @plsc.parallel_loop(0, rows, unroll=4)
    def body(r):
        for j in range(vec16s):
            ref[r, pl.ds(j * L, L)] = v


def _sc_propagate(n, e, nb, nt, nch, ch):
    vq = DQ // L

    mesh = plsc.VectorSubcoreMesh(core_axis_name="c", subcore_axis_name="s")

    @functools.partial(
        pl.kernel,
        mesh=mesh,
        compiler_params=pltpu.CompilerParams(
            use_tc_tiling_on_sc=False, needs_layout_passes=False),
        out_type=(
            jax.ShapeDtypeStruct((n, 4 * DQ), jnp.float32),
            jax.ShapeDtypeStruct((4 * n, DQ), jnp.float32),
        ),
        scratch_types=dict(
            uacc=pltpu.VMEM_SHARED((n, DQ), jnp.float32),
            dga=pltpu.VMEM_SHARED((n, L), jnp.float32),
            srcg0=pltpu.VMEM((nb, EB), jnp.int32),
            dstv=pltpu.VMEM((nb, EB), jnp.int32),
            gbuf0=pltpu.VMEM((EB, DQ), jnp.float32),
            gbuf1=pltpu.VMEM((EB, DQ), jnp.float32),
            gbuf2=pltpu.VMEM((EB, DQ), jnp.float32),
            gbuf3=pltpu.VMEM((EB, DQ), jnp.float32),
            gbuf4=pltpu.VMEM((EB, DQ), jnp.float32),
            ones=pltpu.VMEM((EB, L), jnp.float32),
            nvec=pltpu.VMEM((nt, L), jnp.float32),
            snorm=pltpu.SMEM((nt,), jnp.float32),
            dnorm=pltpu.SMEM((nt,), jnp.float32),
            uch=pltpu.VMEM((ch, DQ), jnp.float32),
            ych=pltpu.VMEM((ch, DQ), jnp.float32),
            ych2=pltpu.VMEM((ch, DQ), jnp.float32),
            gs0=pltpu.SemaphoreType.DMA,
            gs1=pltpu.SemaphoreType.DMA,
            gs2=pltpu.SemaphoreType.DMA,
            gs3=pltpu.SemaphoreType.DMA,
            gs4=pltpu.SemaphoreType.DMA,
            ss0=pltpu.SemaphoreType.DMA,
            ss1=pltpu.SemaphoreType.DMA,
            ss2=pltpu.SemaphoreType.DMA,
            ss3=pltpu.SemaphoreType.DMA,
            ss4=pltpu.SemaphoreType.DMA,
        ),
    )
    def prop(y0_hbm, src_hbm, dst_hbm, out_hbm, h_hbm,
             uacc, dga, srcg0, dstv, gbuf0, gbuf1, gbuf2, gbuf3, gbuf4,
             ones, nvec, snorm, dnorm, uch, ych, ych2,
             gs0, gs1, gs2, gs3, gs4, ss0, ss1, ss2, ss3, ss4):
        c = lax.axis_index("c")
        s = lax.axis_index("s")
        n0 = s * nt

        _fill(ych, ch, vq, 0.0)
        for t in range(nch):
            pltpu.sync_copy(ych, uacc.at[pl.ds(n0 + t * ch, ch)])
        _fill(nvec, nt, 1, 0.0)
        pltpu.sync_copy(nvec, dga.at[pl.ds(n0, nt)])
        _fill(ones, EB, 1, 1.0)
        plsc.subcore_barrier()

        pltpu.sync_copy(src_hbm.at[pl.ds(s * nb, nb)], srcg0)
        pltpu.sync_copy(dst_hbm.at[pl.ds(s * nb, nb)], dstv)
        lag = 6

        def deg_scatter(idx):
            def deg_wait():
                pltpu.make_async_copy(ones, dga.at[idx.at[0]], ss0).wait()

            def deg_body(j, _):
                pltpu.async_copy(ones, dga.at[idx.at[j]], ss0, add=True)

                @pl.when(j >= lag)
                def _w():
                    deg_wait()

                return _

            lax.fori_loop(0, nb, deg_body, None)
            for _ in range(lag):
                deg_wait()

        def _shift(delta):
            dv = jnp.full((L,), delta, jnp.int32)

            @plsc.parallel_loop(0, nb, unroll=4)
            def body(j):
                for v in range(EB // L):
                    sl = pl.ds(v * L, L)
                    srcg0[j, sl] = srcg0[j, sl] + dv

        deg_scatter(srcg0)
        _shift(2 * c * n)
        plsc.subcore_barrier()

        def norms_to(dst1d):
            @plsc.parallel_loop(0, nt, unroll=2)
            def norm_body(r):
                y = _rsqrt16(jnp.maximum(nvec[r, pl.ds(0, L)], 1.0))
                dst1d[r] = y[0]

        pltpu.sync_copy(dga.at[pl.ds(n0, nt)], nvec)
        norms_to(snorm)
        _fill(nvec, nt, 1, 0.0)
        pltpu.sync_copy(nvec, dga.at[pl.ds(n0, nt)])
        plsc.subcore_barrier()
        deg_scatter(dstv)
        plsc.subcore_barrier()
        pltpu.sync_copy(dga.at[pl.ds(n0, nt)], nvec)
        norms_to(dnorm)

        ybufs = (ych, ych2)
        ysem = (gs0, gs1)
        stsem = (ss1, ss2)

        def yq_of(i):
            return (2 * c + (i // nch)) * n + n0 + (i % nch) * ch

        pend_st = [None, None]
        pend_y = [None, None]
        pend_y[0] = pltpu.async_copy(
            y0_hbm.at[pl.ds(yq_of(0), ch)], ybufs[0], ysem[0])
        for i in range(2 * nch):
            qq = i % 2
            yb = ybufs[qq]
            ti = i % nch
            pend_y[qq].wait()
            if i + 1 < 2 * nch:
                q2 = (i + 1) % 2
                if pend_st[q2] is not None:
                    pend_st[q2].wait()
                pend_y[q2] = pltpu.async_copy(
                    y0_hbm.at[pl.ds(yq_of(i + 1), ch)], ybufs[q2], ysem[q2])

            @plsc.parallel_loop(0, ch, unroll=4)
            def h0_body(r):
                sn = snorm[ti * ch + r]
                for v in range(vq):
                    sl = pl.ds(v * L, L)
                    yb[r, sl] = yb[r, sl] * sn

            pend_st[qq] = pltpu.async_copy(
                yb, h_hbm.at[pl.ds(yq_of(i), ch)], stsem[qq])
        for qq in range(2):
            if pend_st[qq] is not None:
                pend_st[qq].wait()
        plsc.subcore_barrier()

        for k in range(K_STEPS):
            last = k == K_STEPS - 1
            for p in range(2):
                if p == 1:
                    _shift(n)

                nbuf = 5
                bufs = (gbuf0, gbuf1, gbuf2, gbuf3, gbuf4)
                gss = (gs0, gs1, gs2, gs3, gs4)
                sss = (ss0, ss1, ss2, ss3, ss4)

                def gather(j, q):
                    pltpu.async_copy(h_hbm.at[srcg0.at[j]], bufs[q], gss[q])

                def gwait(q):
                    pltpu.make_async_copy(
                        h_hbm.at[srcg0.at[0]], bufs[q], gss[q]).wait()

                def scat(j, q):
                    pltpu.async_copy(
                        bufs[q], uacc.at[dstv.at[j]], sss[q], add=True)

                def swait(q):
                    pltpu.make_async_copy(
                        bufs[q], uacc.at[dstv.at[0]], sss[q]).wait()

                for q in range(nbuf):
                    gather(q, q)

                def edge_quad(i, _):
                    j = nbuf * i
                    for q in range(nbuf):
                        gwait(q)
                        scat(j + q, q)
                    for q in range(nbuf):
                        swait(q)

                        @pl.when(j + nbuf + q < nb)
                        def _g(jq=j + nbuf + q, q=q):
                            gather(jq, q)

                    return _

                lax.fori_loop(0, nb // nbuf, edge_quad, None)
                for q in range(nb - nbuf * (nb // nbuf)):
                    gwait(q)
                    scat(nbuf * (nb // nbuf) + q, q)
                    swait(q)
                if p == 1:
                    _shift(-n)
                plsc.subcore_barrier()

                yq0 = (2 * c + p) * n + n0
                pend_h = None
                pend_z = [None, None]
                pend_yd = [None, None]
                pend_yd[0] = pltpu.async_copy(
                    y0_hbm.at[pl.ds(yq0, ch)], ybufs[0], ysem[0])
                for t in range(nch):
                    g0 = n0 + t * ch
                    if pend_h is not None:
                        pend_h.wait()
                    pltpu.sync_copy(uacc.at[pl.ds(g0, ch)], uch)
                    pend_yd[t % 2].wait()
                    if t + 1 < nch:
                        q2 = (t + 1) % 2
                        if pend_z[q2] is not None:
                            pend_z[q2].wait()
                            pend_z[q2] = None
                        pend_yd[q2] = pltpu.async_copy(
                            y0_hbm.at[pl.ds(yq0 + (t + 1) * ch, ch)],
                            ybufs[q2], ysem[q2])
                    yb = ybufs[t % 2]

                    @plsc.parallel_loop(0, ch, unroll=4)
                    def dense_body(r):
                        sn = snorm[t * ch + r]
                        dn = dnorm[t * ch + r]
                        if last:
                            a = (1.0 - ALPHA) * dn
                            b = jnp.float32(ALPHA)
                        else:
                            a = (1.0 - ALPHA) * sn * dn
                            b = ALPHA * sn
                        for v in range(vq):
                            sl = pl.ds(v * L, L)
                            uch[r, sl] = a * uch[r, sl] + b * yb[r, sl]
                    if last:
                        pend_h = pltpu.async_copy(
                            uch,
                            out_hbm.at[pl.ds(g0, ch),
                                       pl.ds((2 * c + p) * DQ, DQ)], ss0)
                    else:
                        pend_h = pltpu.async_copy(
                            uch, h_hbm.at[pl.ds(yq0 + t * ch, ch)], ss0)
                    if not (last and p == 1):
                        _fill(yb, ch, vq, 0.0)
                        pend_z[t % 2] = pltpu.async_copy(
                            yb, uacc.at[pl.ds(g0, ch)], stsem[t % 2])
                pend_h.wait()
                for q2 in range(2):
                    if pend_z[q2] is not None:
                        pend_z[q2].wait()
                if not (last and p == 1):
                    plsc.subcore_barrier()

    return prop


def _tc_matmul(n, d, bn):
    def body(x_ref, w_ref, o_ref):
        o_ref[...] = lax.dot_general(
            x_ref[...], w_ref[...], (((1,), (1,)), ((), ())),
            preferred_element_type=jnp.float32)

    nblk = n // bn
    return pl.pallas_call(
        body,
        grid=(4, nblk),
        in_specs=[
            pl.BlockSpec((bn, d), lambda q, i: (i, 0)),
            pl.BlockSpec((DQ, d), lambda q, i: (q, 0)),
        ],
        out_specs=pl.BlockSpec((bn, DQ), lambda q, i: (q * nblk + i, 0)),
        out_shape=jax.ShapeDtypeStruct((4 * n, DQ), jnp.float32),
    )


def kernel(x, edge_index, W):
    n, d = x.shape
    e = edge_index.shape[1]

    src = edge_index[0].astype(jnp.int32)
    dst = edge_index[1].astype(jnp.int32)
    nb = e // (NS * EB)
    src2 = src.reshape(NS * nb, EB)
    dst2 = dst.reshape(NS * nb, EB)

    nt = n // NS
    ch = 125
    nch = nt // ch

    y0 = _tc_matmul(n, d, bn=400)(x, W)
    out, _ = _sc_propagate(n, e, nb, nt, nch, ch)(y0, src2, dst2)
    return out


# speedup vs baseline: 6.9572x; 1.1031x over previous
import functools

import jax
import jax.numpy as jnp
from jax import lax
from jax.experimental import pallas as pl
from jax.experimental.pallas import tpu as pltpu
from jax.experimental.pallas import tpu_sc as plsc

K_STEPS = 3
ALPHA = 0.5
NS = 16
NC = 2
L = 16
EB = 80
DQ = 64


def _rsqrt16(d):
    half = d * 0.5
    i = plsc.bitcast(d, jnp.int32)
    i = jnp.full((L,), 0x5F3759DF, jnp.int32) - lax.shift_right_arithmetic(
        i, jnp.full((L,), 1, jnp.int32))
    y = plsc.bitcast(i, jnp.float32)
    for _ in range(3):
        y = y * (1.5 - half * y * y)
    return y


def _fill(ref, rows, vec16s, value):
    v = jnp.full((L,), value, jnp.float32)

    @plsc.parallel_loop(0, rows, unroll=4)
    def body(r):
        for j in range(vec16s):
            ref[r, pl.ds(j * L, L)] = v


def _sc_propagate(n, e, nb, nt, nch, ch):
    vq = DQ // L

    mesh = plsc.VectorSubcoreMesh(core_axis_name="c", subcore_axis_name="s")

    @functools.partial(
        pl.kernel,
        mesh=mesh,
        compiler_params=pltpu.CompilerParams(
            use_tc_tiling_on_sc=False, needs_layout_passes=False),
        out_type=(
            jax.ShapeDtypeStruct((n, 4 * DQ), jnp.float32),
            jax.ShapeDtypeStruct((4 * n, DQ), jnp.float32),
        ),
        scratch_types=dict(
            uacc=pltpu.VMEM_SHARED((n, DQ), jnp.float32),
            dga=pltpu.VMEM_SHARED((n, L), jnp.float32),
            srcg0=pltpu.VMEM((nb, EB), jnp.int32),
            dstv=pltpu.VMEM((nb, EB), jnp.int32),
            gbuf0=pltpu.VMEM((EB, DQ), jnp.float32),
            gbuf1=pltpu.VMEM((EB, DQ), jnp.float32),
            gbuf2=pltpu.VMEM((EB, DQ), jnp.float32),
            gbuf3=pltpu.VMEM((EB, DQ), jnp.float32),
            gbuf4=pltpu.VMEM((EB, DQ), jnp.float32),
            ones=pltpu.VMEM((EB, L), jnp.float32),
            nvec=pltpu.VMEM((nt, L), jnp.float32),
            snorm=pltpu.SMEM((nt,), jnp.float32),
            dnorm=pltpu.SMEM((nt,), jnp.float32),
            uch=pltpu.VMEM((ch, DQ), jnp.float32),
            ych=pltpu.VMEM((ch, DQ), jnp.float32),
            ych2=pltpu.VMEM((ch, DQ), jnp.float32),
            gs0=pltpu.SemaphoreType.DMA,
            gs1=pltpu.SemaphoreType.DMA,
            gs2=pltpu.SemaphoreType.DMA,
            gs3=pltpu.SemaphoreType.DMA,
            gs4=pltpu.SemaphoreType.DMA,
            ss0=pltpu.SemaphoreType.DMA,
            ss1=pltpu.SemaphoreType.DMA,
            ss2=pltpu.SemaphoreType.DMA,
            ss3=pltpu.SemaphoreType.DMA,
            ss4=pltpu.SemaphoreType.DMA,
        ),
    )
    def prop(y0_hbm, edges_hbm, out_hbm, h_hbm,
             uacc, dga, srcg0, dstv, gbuf0, gbuf1, gbuf2, gbuf3, gbuf4,
             ones, nvec, snorm, dnorm, uch, ych, ych2,
             gs0, gs1, gs2, gs3, gs4, ss0, ss1, ss2, ss3, ss4):
        c = lax.axis_index("c")
        s = lax.axis_index("s")
        n0 = s * nt

        _fill(ych, ch, vq, 0.0)
        for t in range(nch):
            pltpu.sync_copy(ych, uacc.at[pl.ds(n0 + t * ch, ch)])
        _fill(nvec, nt, 1, 0.0)
        pltpu.sync_copy(nvec, dga.at[pl.ds(n0, nt)])
        _fill(ones, EB, 1, 1.0)
        plsc.subcore_barrier()

        pltpu.sync_copy(edges_hbm.at[0, pl.ds(s * nb, nb)], srcg0)
        pltpu.sync_copy(edges_hbm.at[1, pl.ds(s * nb, nb)], dstv)
        lag = 6

        def deg_scatter(idx):
            def deg_wait():
                pltpu.make_async_copy(ones, dga.at[idx.at[0]], ss0).wait()

            def deg_body(j, _):
                pltpu.async_copy(ones, dga.at[idx.at[j]], ss0, add=True)

                @pl.when(j >= lag)
                def _w():
                    deg_wait()

                return _

            lax.fori_loop(0, nb, deg_body, None)
            for _ in range(lag):
                deg_wait()

        def _shift(delta):
            dv = jnp.full((L,), delta, jnp.int32)

            @plsc.parallel_loop(0, nb, unroll=4)
            def body(j):
                for v in range(EB // L):
                    sl = pl.ds(v * L, L)
                    srcg0[j, sl] = srcg0[j, sl] + dv

        deg_scatter(srcg0)
        _shift(2 * c * n)
        plsc.subcore_barrier()

        def norms_to(dst1d):
            @plsc.parallel_loop(0, nt, unroll=2)
            def norm_body(r):
                y = _rsqrt16(jnp.maximum(nvec[r, pl.ds(0, L)], 1.0))
                dst1d[r] = y[0]

        pltpu.sync_copy(dga.at[pl.ds(n0, nt)], nvec)
        norms_to(snorm)
        _fill(nvec, nt, 1, 0.0)
        pltpu.sync_copy(nvec, dga.at[pl.ds(n0, nt)])
        plsc.subcore_barrier()
        deg_scatter(dstv)
        plsc.subcore_barrier()
        pltpu.sync_copy(dga.at[pl.ds(n0, nt)], nvec)
        norms_to(dnorm)

        ybufs = (ych, ych2)
        ysem = (gs0, gs1)
        stsem = (ss1, ss2)

        def yq_of(i):
            return (2 * c + (i // nch)) * n + n0 + (i % nch) * ch

        pend_st = [None, None]
        pend_y = [None, None]
        pend_y[0] = pltpu.async_copy(
            y0_hbm.at[pl.ds(yq_of(0), ch)], ybufs[0], ysem[0])
        for i in range(2 * nch):
            qq = i % 2
            yb = ybufs[qq]
            ti = i % nch
            pend_y[qq].wait()
            if i + 1 < 2 * nch:
                q2 = (i + 1) % 2
                if pend_st[q2] is not None:
                    pend_st[q2].wait()
                pend_y[q2] = pltpu.async_copy(
                    y0_hbm.at[pl.ds(yq_of(i + 1), ch)], ybufs[q2], ysem[q2])

            @plsc.parallel_loop(0, ch, unroll=4)
            def h0_body(r):
                sn = snorm[ti * ch + r]
                for v in range(vq):
                    sl = pl.ds(v * L, L)
                    yb[r, sl] = yb[r, sl] * sn

            pend_st[qq] = pltpu.async_copy(
                yb, h_hbm.at[pl.ds(yq_of(i), ch)], stsem[qq])
        for qq in range(2):
            if pend_st[qq] is not None:
                pend_st[qq].wait()
        plsc.subcore_barrier()

        for k in range(K_STEPS):
            last = k == K_STEPS - 1
            for p in range(2):
                if p == 1:
                    _shift(n)

                nbuf = 5
                bufs = (gbuf0, gbuf1, gbuf2, gbuf3, gbuf4)
                gss = (gs0, gs1, gs2, gs3, gs4)
                sss = (ss0, ss1, ss2, ss3, ss4)

                def gather(j, q):
                    pltpu.async_copy(h_hbm.at[srcg0.at[j]], bufs[q], gss[q])

                def gwait(q):
                    pltpu.make_async_copy(
                        h_hbm.at[srcg0.at[0]], bufs[q], gss[q]).wait()

                def scat(j, q):
                    pltpu.async_copy(
                        bufs[q], uacc.at[dstv.at[j]], sss[q], add=True)

                def swait(q):
                    pltpu.make_async_copy(
                        bufs[q], uacc.at[dstv.at[0]], sss[q]).wait()

                for q in range(nbuf):
                    gather(q, q)

                def edge_quad(i, _):
                    j = nbuf * i
                    for q in range(nbuf):
                        gwait(q)
                        scat(j + q, q)
                    for q in range(nbuf):
                        swait(q)

                        @pl.when(j + nbuf + q < nb)
                        def _g(jq=j + nbuf + q, q=q):
                            gather(jq, q)

                    return _

                lax.fori_loop(0, nb // nbuf, edge_quad, None)
                for q in range(nb - nbuf * (nb // nbuf)):
                    gwait(q)
                    scat(nbuf * (nb // nbuf) + q, q)
                    swait(q)
                if p == 1:
                    _shift(-n)
                plsc.subcore_barrier()

                yq0 = (2 * c + p) * n + n0
                pend_h = None
                pend_z = [None, None]
                pend_yd = [None, None]
                pend_yd[0] = pltpu.async_copy(
                    y0_hbm.at[pl.ds(yq0, ch)], ybufs[0], ysem[0])
                for t in range(nch):
                    g0 = n0 + t * ch
                    if pend_h is not None:
                        pend_h.wait()
                    pltpu.sync_copy(uacc.at[pl.ds(g0, ch)], uch)
                    pend_yd[t % 2].wait()
                    if t + 1 < nch:
                        q2 = (t + 1) % 2
                        if pend_z[q2] is not None:
                            pend_z[q2].wait()
                            pend_z[q2] = None
                        pend_yd[q2] = pltpu.async_copy(
                            y0_hbm.at[pl.ds(yq0 + (t + 1) * ch, ch)],
                            ybufs[q2], ysem[q2])
                    yb = ybufs[t % 2]

                    @plsc.parallel_loop(0, ch, unroll=4)
                    def dense_body(r):
                        sn = snorm[t * ch + r]
                        dn = dnorm[t * ch + r]
                        if last:
                            a = (1.0 - ALPHA) * dn
                            b = jnp.float32(ALPHA)
                        else:
                            a = (1.0 - ALPHA) * sn * dn
                            b = ALPHA * sn
                        for v in range(vq):
                            sl = pl.ds(v * L, L)
                            uch[r, sl] = a * uch[r, sl] + b * yb[r, sl]
                    if last:
                        pend_h = pltpu.async_copy(
                            uch,
                            out_hbm.at[pl.ds(g0, ch),
                                       pl.ds((2 * c + p) * DQ, DQ)], ss0)
                    else:
                        pend_h = pltpu.async_copy(
                            uch, h_hbm.at[pl.ds(yq0 + t * ch, ch)], ss0)
                    if not (last and p == 1):
                        _fill(yb, ch, vq, 0.0)
                        pend_z[t % 2] = pltpu.async_copy(
                            yb, uacc.at[pl.ds(g0, ch)], stsem[t % 2])
                pend_h.wait()
                for q2 in range(2):
                    if pend_z[q2] is not None:
                        pend_z[q2].wait()
                if not (last and p == 1):
                    plsc.subcore_barrier()

    return prop


def _tc_matmul(n, d, bn):
    def body(x_ref, w_ref, o_ref):
        o_ref[...] = lax.dot_general(
            x_ref[...], w_ref[...], (((1,), (1,)), ((), ())),
            preferred_element_type=jnp.float32)

    nblk = n // bn
    return pl.pallas_call(
        body,
        grid=(4, nblk),
        in_specs=[
            pl.BlockSpec((bn, d), lambda q, i: (i, 0)),
            pl.BlockSpec((DQ, d), lambda q, i: (q, 0)),
        ],
        out_specs=pl.BlockSpec((bn, DQ), lambda q, i: (q * nblk + i, 0)),
        out_shape=jax.ShapeDtypeStruct((4 * n, DQ), jnp.float32),
    )


def kernel(x, edge_index, W):
    n, d = x.shape
    e = edge_index.shape[1]

    nb = e // (NS * EB)
    edges = edge_index.astype(jnp.int32).reshape(2, NS * nb, EB)

    nt = n // NS
    ch = 125
    nch = nt // ch

    y0 = _tc_matmul(n, d, bn=2000)(x, W)
    out, _ = _sc_propagate(n, e, nb, nt, nch, ch)(y0, edges)
    return out


# speedup vs baseline: 6.9608x; 1.0005x over previous
import functools

import jax
import jax.numpy as jnp
from jax import lax
from jax.experimental import pallas as pl
from jax.experimental.pallas import tpu as pltpu
from jax.experimental.pallas import tpu_sc as plsc

K_STEPS = 3
ALPHA = 0.5
NS = 16
NC = 2
L = 16
EB = 80
DQ = 64


def _rsqrt16(d):
    half = d * 0.5
    i = plsc.bitcast(d, jnp.int32)
    i = jnp.full((L,), 0x5F3759DF, jnp.int32) - lax.shift_right_arithmetic(
        i, jnp.full((L,), 1, jnp.int32))
    y = plsc.bitcast(i, jnp.float32)
    for _ in range(3):
        y = y * (1.5 - half * y * y)
    return y


def _fill(ref, rows, vec16s, value):
    v = jnp.full((L,), value, jnp.float32)

    @plsc.parallel_loop(0, rows, unroll=4)
    def body(r):
        for j in range(vec16s):
            ref[r, pl.ds(j * L, L)] = v


def _sc_propagate(n, e, nb, nt, nch, ch):
    vq = DQ // L

    mesh = plsc.VectorSubcoreMesh(core_axis_name="c", subcore_axis_name="s")

    @functools.partial(
        pl.kernel,
        mesh=mesh,
        compiler_params=pltpu.CompilerParams(
            use_tc_tiling_on_sc=False, needs_layout_passes=False),
        out_type=(
            jax.ShapeDtypeStruct((n, 4 * DQ), jnp.float32),
            jax.ShapeDtypeStruct((4 * n, DQ), jnp.float32),
        ),
        scratch_types=dict(
            uacc=pltpu.VMEM_SHARED((n, DQ), jnp.float32),
            dga=pltpu.VMEM_SHARED((n, L), jnp.float32),
            srcg0=pltpu.VMEM((nb, EB), jnp.int32),
            dstv=pltpu.VMEM((nb, EB), jnp.int32),
            gbuf0=pltpu.VMEM((EB, DQ), jnp.float32),
            gbuf1=pltpu.VMEM((EB, DQ), jnp.float32),
            gbuf2=pltpu.VMEM((EB, DQ), jnp.float32),
            gbuf3=pltpu.VMEM((EB, DQ), jnp.float32),
            gbuf4=pltpu.VMEM((EB, DQ), jnp.float32),
            ones=pltpu.VMEM((EB, L), jnp.float32),
            nvec=pltpu.VMEM((nt, L), jnp.float32),
            snorm=pltpu.SMEM((nt,), jnp.float32),
            dnorm=pltpu.SMEM((nt,), jnp.float32),
            uch=pltpu.VMEM((ch, DQ), jnp.float32),
            ych=pltpu.VMEM((ch, DQ), jnp.float32),
            ych2=pltpu.VMEM((ch, DQ), jnp.float32),
            gs0=pltpu.SemaphoreType.DMA,
            gs1=pltpu.SemaphoreType.DMA,
            gs2=pltpu.SemaphoreType.DMA,
            gs3=pltpu.SemaphoreType.DMA,
            gs4=pltpu.SemaphoreType.DMA,
            ss0=pltpu.SemaphoreType.DMA,
            ss1=pltpu.SemaphoreType.DMA,
            ss2=pltpu.SemaphoreType.DMA,
            ss3=pltpu.SemaphoreType.DMA,
            ss4=pltpu.SemaphoreType.DMA,
        ),
    )
    def prop(y0_hbm, edges_hbm, out_hbm, h_hbm,
             uacc, dga, srcg0, dstv, gbuf0, gbuf1, gbuf2, gbuf3, gbuf4,
             ones, nvec, snorm, dnorm, uch, ych, ych2,
             gs0, gs1, gs2, gs3, gs4, ss0, ss1, ss2, ss3, ss4):
        c = lax.axis_index("c")
        s = lax.axis_index("s")
        n0 = s * nt

        _fill(ych, ch, vq, 0.0)
        for t in range(nch):
            pltpu.sync_copy(ych, uacc.at[pl.ds(n0 + t * ch, ch)])
        _fill(nvec, nt, 1, 0.0)
        pltpu.sync_copy(nvec, dga.at[pl.ds(n0, nt)])
        _fill(ones, EB, 1, 1.0)
        plsc.subcore_barrier()

        pltpu.sync_copy(edges_hbm.at[0, pl.ds(s * nb, nb)], srcg0)
        pltpu.sync_copy(edges_hbm.at[1, pl.ds(s * nb, nb)], dstv)
        lag = 10

        def deg_scatter(idx):
            def deg_wait():
                pltpu.make_async_copy(ones, dga.at[idx.at[0]], ss0).wait()

            def deg_body(j, _):
                pltpu.async_copy(ones, dga.at[idx.at[j]], ss0, add=True)

                @pl.when(j >= lag)
                def _w():
                    deg_wait()

                return _

            lax.fori_loop(0, nb, deg_body, None)
            for _ in range(lag):
                deg_wait()

        def _shift(delta):
            dv = jnp.full((L,), delta, jnp.int32)

            @plsc.parallel_loop(0, nb, unroll=4)
            def body(j):
                for v in range(EB // L):
                    sl = pl.ds(v * L, L)
                    srcg0[j, sl] = srcg0[j, sl] + dv

        deg_scatter(srcg0)
        _shift(2 * c * n)
        plsc.subcore_barrier()

        def norms_to(dst1d):
            @plsc.parallel_loop(0, nt, unroll=2)
            def norm_body(r):
                y = _rsqrt16(jnp.maximum(nvec[r, pl.ds(0, L)], 1.0))
                dst1d[r] = y[0]

        pltpu.sync_copy(dga.at[pl.ds(n0, nt)], nvec)
        norms_to(snorm)
        _fill(nvec, nt, 1, 0.0)
        pltpu.sync_copy(nvec, dga.at[pl.ds(n0, nt)])
        plsc.subcore_barrier()
        deg_scatter(dstv)
        plsc.subcore_barrier()
        pltpu.sync_copy(dga.at[pl.ds(n0, nt)], nvec)
        norms_to(dnorm)

        ybufs = (ych, ych2)
        ysem = (gs0, gs1)
        stsem = (ss1, ss2)

        def yq_of(i):
            return (2 * c + (i // nch)) * n + n0 + (i % nch) * ch

        pend_st = [None, None]
        pend_y = [None, None]
        pend_y[0] = pltpu.async_copy(
            y0_hbm.at[pl.ds(yq_of(0), ch)], ybufs[0], ysem[0])
        for i in range(2 * nch):
            qq = i % 2
            yb = ybufs[qq]
            ti = i % nch
            pend_y[qq].wait()
            if i + 1 < 2 * nch:
                q2 = (i + 1) % 2
                if pend_st[q2] is not None:
                    pend_st[q2].wait()
                pend_y[q2] = pltpu.async_copy(
                    y0_hbm.at[pl.ds(yq_of(i + 1), ch)], ybufs[q2], ysem[q2])

            @plsc.parallel_loop(0, ch, unroll=4)
            def h0_body(r):
                sn = snorm[ti * ch + r]
                for v in range(vq):
                    sl = pl.ds(v * L, L)
                    yb[r, sl] = yb[r, sl] * sn

            pend_st[qq] = pltpu.async_copy(
                yb, h_hbm.at[pl.ds(yq_of(i), ch)], stsem[qq])
        for qq in range(2):
            if pend_st[qq] is not None:
                pend_st[qq].wait()
        plsc.subcore_barrier()

        for k in range(K_STEPS):
            last = k == K_STEPS - 1
            for p in range(2):
                if p == 1:
                    _shift(n)

                nbuf = 5
                bufs = (gbuf0, gbuf1, gbuf2, gbuf3, gbuf4)
                gss = (gs0, gs1, gs2, gs3, gs4)
                sss = (ss0, ss1, ss2, ss3, ss4)

                def gather(j, q):
                    pltpu.async_copy(h_hbm.at[srcg0.at[j]], bufs[q], gss[q])

                def gwait(q):
                    pltpu.make_async_copy(
                        h_hbm.at[srcg0.at[0]], bufs[q], gss[q]).wait()

                def scat(j, q):
                    pltpu.async_copy(
                        bufs[q], uacc.at[dstv.at[j]], sss[q], add=True)

                def swait(q):
                    pltpu.make_async_copy(
                        bufs[q], uacc.at[dstv.at[0]], sss[q]).wait()

                for q in range(nbuf):
                    gather(q, q)

                def edge_quad(i, _):
                    j = nbuf * i
                    for q in range(nbuf):
                        gwait(q)
                        scat(j + q, q)
                    for q in range(nbuf):
                        swait(q)

                        @pl.when(j + nbuf + q < nb)
                        def _g(jq=j + nbuf + q, q=q):
                            gather(jq, q)

                    return _

                lax.fori_loop(0, nb // nbuf, edge_quad, None)
                for q in range(nb - nbuf * (nb // nbuf)):
                    gwait(q)
                    scat(nbuf * (nb // nbuf) + q, q)
                    swait(q)
                if p == 1:
                    _shift(-n)
                plsc.subcore_barrier()

                yq0 = (2 * c + p) * n + n0
                pend_h = None
                pend_z = [None, None]
                pend_yd = [None, None]
                pend_yd[0] = pltpu.async_copy(
                    y0_hbm.at[pl.ds(yq0, ch)], ybufs[0], ysem[0])
                for t in range(nch):
                    g0 = n0 + t * ch
                    if pend_h is not None:
                        pend_h.wait()
                    pltpu.sync_copy(uacc.at[pl.ds(g0, ch)], uch)
                    pend_yd[t % 2].wait()
                    if t + 1 < nch:
                        q2 = (t + 1) % 2
                        if pend_z[q2] is not None:
                            pend_z[q2].wait()
                            pend_z[q2] = None
                        pend_yd[q2] = pltpu.async_copy(
                            y0_hbm.at[pl.ds(yq0 + (t + 1) * ch, ch)],
                            ybufs[q2], ysem[q2])
                    yb = ybufs[t % 2]

                    @plsc.parallel_loop(0, ch, unroll=4)
                    def dense_body(r):
                        sn = snorm[t * ch + r]
                        dn = dnorm[t * ch + r]
                        if last:
                            a = (1.0 - ALPHA) * dn
                            b = jnp.float32(ALPHA)
                        else:
                            a = (1.0 - ALPHA) * sn * dn
                            b = ALPHA * sn
                        for v in range(vq):
                            sl = pl.ds(v * L, L)
                            uch[r, sl] = a * uch[r, sl] + b * yb[r, sl]
                    if last:
                        pend_h = pltpu.async_copy(
                            uch,
                            out_hbm.at[pl.ds(g0, ch),
                                       pl.ds((2 * c + p) * DQ, DQ)], ss0)
                    else:
                        pend_h = pltpu.async_copy(
                            uch, h_hbm.at[pl.ds(yq0 + t * ch, ch)], ss0)
                    if not (last and p == 1):
                        _fill(yb, ch, vq, 0.0)
                        pend_z[t % 2] = pltpu.async_copy(
                            yb, uacc.at[pl.ds(g0, ch)], stsem[t % 2])
                pend_h.wait()
                for q2 in range(2):
                    if pend_z[q2] is not None:
                        pend_z[q2].wait()
                if not (last and p == 1):
                    plsc.subcore_barrier()

    return prop


def _tc_matmul(n, d, bn):
    def body(x_ref, w_ref, o_ref):
        o_ref[...] = lax.dot_general(
            x_ref[...], w_ref[...], (((1,), (1,)), ((), ())),
            preferred_element_type=jnp.float32)

    nblk = n // bn
    return pl.pallas_call(
        body,
        grid=(4, nblk),
        in_specs=[
            pl.BlockSpec((bn, d), lambda q, i: (i, 0)),
            pl.BlockSpec((DQ, d), lambda q, i: (q, 0)),
        ],
        out_specs=pl.BlockSpec((bn, DQ), lambda q, i: (q * nblk + i, 0)),
        out_shape=jax.ShapeDtypeStruct((4 * n, DQ), jnp.float32),
    )


def kernel(x, edge_index, W):
    n, d = x.shape
    e = edge_index.shape[1]

    nb = e // (NS * EB)
    edges = edge_index.astype(jnp.int32).reshape(2, NS * nb, EB)

    nt = n // NS
    ch = 125
    nch = nt // ch

    y0 = _tc_matmul(n, d, bn=2000)(x, W)
    out, _ = _sc_propagate(n, e, nb, nt, nch, ch)(y0, edges)
    return out
